# baseline (speedup 1.0000x reference)
import jax
import jax.numpy as jnp
from jax import lax
from jax.experimental import pallas as pl
from jax.experimental.pallas import tpu as pltpu
from jax.experimental.pallas import tpu_sc as plsc

X, Y, Z = 192, 160, 192
N = 300
NUM_CORES = 2
NUM_TILES = 16
NUM_WORKERS = NUM_CORES * NUM_TILES
PER_TILE = 16
NPAD = NUM_WORKERS * PER_TILE
L = 16
G = 3 * X * (Y // 8)
CHUNKS = 3 * PER_TILE

OFF_FX, OFF_FY, OFF_FZ = 0, 16, 32
OFF_MX, OFF_MY, OFF_MZ = 48, 64, 80
OFF_SP = 96
PACK_W = 256


def _tre_body(f3_hbm, pk_hbm, out_hbm,
              pk_v, buf_v, part_v, all_v, out_v, shared, sem):
    c = lax.axis_index("c")
    s = lax.axis_index("s")
    wid = c * NUM_TILES + s

    pltpu.sync_copy(pk_hbm.at[pl.ds(wid * PACK_W, PACK_W)], pk_v)

    fxj = pk_v[pl.ds(OFF_FX, L)]
    fyj = pk_v[pl.ds(OFF_FY, L)]
    fzj = pk_v[pl.ds(OFF_FZ, L)]
    g0 = fxj * (Y // 8) + jnp.right_shift(fyj, 3)
    iyj = jnp.bitwise_and(fyj, 7)
    zcj = jnp.right_shift(fzj, 7)
    copies = []
    for i in range(L):
        g = g0[i]
        iyi = iyj[i]
        zoff = zcj[i] * 128
        for ch in range(3):
            copies.append(pltpu.async_copy(
                f3_hbm.at[g + ch * (X * Y // 8), iyi, pl.ds(zoff, 128)],
                buf_v.at[ch * PER_TILE + i], sem))
    for cp in copies:
        cp.wait()

    sx = plsc.bitcast(pk_v[pl.ds(OFF_SP, L)], jnp.float32)
    sy = plsc.bitcast(pk_v[pl.ds(OFF_SP + L, L)], jnp.float32)
    sz = plsc.bitcast(pk_v[pl.ds(OFF_SP + 2 * L, L)], jnp.float32)
    mxj = plsc.bitcast(pk_v[pl.ds(OFF_MX, L)], jnp.float32)
    myj = plsc.bitcast(pk_v[pl.ds(OFF_MY, L)], jnp.float32)
    mzj = plsc.bitcast(pk_v[pl.ds(OFF_MZ, L)], jnp.float32)
    lanes = jnp.bitwise_and(fzj, 127)
    rows0 = lax.iota(jnp.int32, L)
    dispx = plsc.load_gather(buf_v, [rows0, lanes])
    dispy = plsc.load_gather(buf_v, [rows0 + PER_TILE, lanes])
    dispz = plsc.load_gather(buf_v, [rows0 + 2 * PER_TILE, lanes])
    dx = (fxj.astype(jnp.float32) + dispx - mxj) * sx
    dy = (fyj.astype(jnp.float32) + dispy - myj) * sy
    dz = (fzj.astype(jnp.float32) + dispz - mzj) * sz
    d2 = dx * dx + dy * dy + dz * dz
    n_global = wid * L + rows0
    acc = jnp.where(n_global < N, d2, 0.0)

    part_v[...] = acc
    pltpu.sync_copy(part_v, shared.at[pl.ds(s * L, L)])
    plsc.subcore_barrier()

    @pl.when(s == 0)
    def _():
        pltpu.sync_copy(shared, all_v)
        tot = jnp.zeros((L,), jnp.float32)
        for r in range(NUM_TILES):
            tot = tot + all_v[pl.ds(r * L, L)]
        total = tot[0]
        for i in range(1, L):
            total = total + tot[i]
        out_v[...] = jnp.full((L,), total * (1.0 / N), jnp.float32)
        pltpu.sync_copy(out_v, out_hbm.at[pl.ds(c * 32, L)])


@jax.jit
def _tre(f3, pk):
    mesh = plsc.VectorSubcoreMesh(core_axis_name="c", subcore_axis_name="s")
    run = pl.kernel(
        _tre_body,
        out_type=jax.ShapeDtypeStruct((64,), jnp.float32),
        mesh=mesh,
        scratch_types=[
            pltpu.VMEM((PACK_W,), jnp.int32),
            pltpu.VMEM((CHUNKS, 128), jnp.float32),
            pltpu.VMEM((L,), jnp.float32),
            pltpu.VMEM((NUM_TILES * L,), jnp.float32),
            pltpu.VMEM((L,), jnp.float32),
            pltpu.VMEM_SHARED((NUM_TILES * L,), jnp.float32),
            pltpu.SemaphoreType.DMA,
        ],
        compiler_params=pltpu.CompilerParams(
            use_tc_tiling_on_sc=True, needs_layout_passes=False),
    )
    return run(f3, pk)


def kernel(vector_field, moving_landmarks, fixed_landmarks, image_spacing):
    f3 = vector_field.reshape(G, 8, Z)
    fl = fixed_landmarks[0].astype(jnp.int32)
    mlb = jax.lax.bitcast_convert_type(moving_landmarks[0], jnp.int32)
    pad = NPAD - N
    flp = jnp.pad(fl, ((0, pad), (0, 0)))
    flp = flp.reshape(NUM_WORKERS, PER_TILE, 3).transpose(0, 2, 1)
    flp = flp.reshape(NUM_WORKERS, 3 * PER_TILE)
    mlp = jnp.pad(mlb, ((0, pad), (0, 0)))
    mlp = mlp.reshape(NUM_WORKERS, PER_TILE, 3).transpose(0, 2, 1)
    mlp = mlp.reshape(NUM_WORKERS, 3 * PER_TILE)
    spb = jnp.repeat(
        jax.lax.bitcast_convert_type(image_spacing.astype(jnp.float32),
                                     jnp.int32), L)
    spt = jnp.broadcast_to(spb, (NUM_WORKERS, 3 * L))
    padw = jnp.zeros((NUM_WORKERS, PACK_W - OFF_SP - 3 * L), jnp.int32)
    pk = jnp.concatenate([flp, mlp, spt, padw], axis=1)
    pk = pk.reshape(NUM_WORKERS * PACK_W)
    out = _tre(f3, pk)
    return out[0] + out[32]


# speedup vs baseline: 1.1802x; 1.1802x over previous
import jax
import jax.numpy as jnp
from jax import lax
from jax.experimental import pallas as pl
from jax.experimental.pallas import tpu as pltpu
from jax.experimental.pallas import tpu_sc as plsc

X, Y, Z = 192, 160, 192
N = 300
NUM_CORES = 2
NUM_TILES = 16
NUM_WORKERS = NUM_CORES * NUM_TILES
SLOTS = 10
FULL_SLOTS = 9
REM = N - FULL_SLOTS * NUM_WORKERS
L = 16
G = 3 * X * (Y // 8)
CHUNKS = 3 * L
PKI_W = 3 * L
PKF_W = 3 * L


def _tre_body(f3_hbm, pki_hbm, pkf_hbm, sp_hbm, out_hbm,
              pki_v, pkf_v, sp_v, buf_v, part_v, all_v, out_v, shared, sem):
    c = lax.axis_index("c")
    s = lax.axis_index("s")
    wid = c * NUM_TILES + s

    pltpu.sync_copy(pki_hbm.at[pl.ds(wid * PKI_W, PKI_W)], pki_v)
    pltpu.sync_copy(pkf_hbm.at[pl.ds(wid * PKF_W, PKF_W)], pkf_v)
    pltpu.sync_copy(sp_hbm, sp_v)

    fxj = pki_v[pl.ds(0, L)]
    fyj = pki_v[pl.ds(L, L)]
    fzj = pki_v[pl.ds(2 * L, L)]
    g0 = fxj * (Y // 8) + jnp.right_shift(fyj, 3)
    iyj = jnp.bitwise_and(fyj, 7)
    zcj = jnp.right_shift(fzj, 7)
    copies = []
    for i in range(FULL_SLOTS):
        g = g0[i]
        iyi = iyj[i]
        zoff = zcj[i] * 128
        for ch in range(3):
            copies.append(pltpu.async_copy(
                f3_hbm.at[g + ch * (X * Y // 8), iyi, pl.ds(zoff, 128)],
                buf_v.at[ch * L + i], sem))

    @pl.when(wid < REM)
    def _():
        i = FULL_SLOTS
        g = g0[i]
        iyi = iyj[i]
        zoff = zcj[i] * 128
        for ch in range(3):
            pltpu.async_copy(
                f3_hbm.at[g + ch * (X * Y // 8), iyi, pl.ds(zoff, 128)],
                buf_v.at[ch * L + i], sem).wait()

    for cp in copies:
        cp.wait()

    sx = sp_v[pl.ds(0, L)]
    sy = sp_v[pl.ds(L, L)]
    sz = sp_v[pl.ds(2 * L, L)]
    mxj = pkf_v[pl.ds(0, L)]
    myj = pkf_v[pl.ds(L, L)]
    mzj = pkf_v[pl.ds(2 * L, L)]
    lanes = jnp.bitwise_and(fzj, 127)
    rows0 = lax.iota(jnp.int32, L)
    dispx = plsc.load_gather(buf_v, [rows0, lanes])
    dispy = plsc.load_gather(buf_v, [rows0 + L, lanes])
    dispz = plsc.load_gather(buf_v, [rows0 + 2 * L, lanes])
    dx = (fxj.astype(jnp.float32) + dispx - mxj) * sx
    dy = (fyj.astype(jnp.float32) + dispy - myj) * sy
    dz = (fzj.astype(jnp.float32) + dispz - mzj) * sz
    d2 = dx * dx + dy * dy + dz * dz
    n_global = rows0 * NUM_WORKERS + wid
    acc = jnp.where(n_global < N, d2, 0.0)

    part_v[...] = acc
    pltpu.sync_copy(part_v, shared.at[pl.ds(s * L, L)])
    plsc.subcore_barrier()

    @pl.when(s == 0)
    def _():
        pltpu.sync_copy(shared, all_v)
        tot = jnp.zeros((L,), jnp.float32)
        for r in range(NUM_TILES):
            tot = tot + all_v[pl.ds(r * L, L)]
        total = tot[0]
        for i in range(1, L):
            total = total + tot[i]
        out_v[...] = jnp.full((L,), total * (1.0 / N), jnp.float32)
        pltpu.sync_copy(out_v, out_hbm.at[pl.ds(c * 32, L)])


@jax.jit
def _tre(f3, pki, pkf, spb):
    mesh = plsc.VectorSubcoreMesh(core_axis_name="c", subcore_axis_name="s")
    run = pl.kernel(
        _tre_body,
        out_type=jax.ShapeDtypeStruct((64,), jnp.float32),
        mesh=mesh,
        scratch_types=[
            pltpu.VMEM((PKI_W,), jnp.int32),
            pltpu.VMEM((PKF_W,), jnp.float32),
            pltpu.VMEM((3 * L,), jnp.float32),
            pltpu.VMEM((CHUNKS, 128), jnp.float32),
            pltpu.VMEM((L,), jnp.float32),
            pltpu.VMEM((NUM_TILES * L,), jnp.float32),
            pltpu.VMEM((L,), jnp.float32),
            pltpu.VMEM_SHARED((NUM_TILES * L,), jnp.float32),
            pltpu.SemaphoreType.DMA,
        ],
        compiler_params=pltpu.CompilerParams(
            use_tc_tiling_on_sc=True, needs_layout_passes=False),
    )
    return run(f3, pki, pkf, spb)


def _pack(arr, dtype):
    pad = SLOTS * NUM_WORKERS - N
    a = jnp.pad(arr, ((0, pad), (0, 0)))
    a = a.reshape(SLOTS, NUM_WORKERS, 3)
    a = a.transpose(1, 2, 0)
    a = jnp.pad(a, ((0, 0), (0, 0), (0, L - SLOTS)))
    return a.reshape(NUM_WORKERS * 3 * L).astype(dtype)


def kernel(vector_field, moving_landmarks, fixed_landmarks, image_spacing):
    f3 = vector_field.reshape(G, 8, Z)
    pki = _pack(fixed_landmarks[0].astype(jnp.int32), jnp.int32)
    pkf = _pack(moving_landmarks[0], jnp.float32)
    spb = jnp.repeat(image_spacing.astype(jnp.float32), L)
    out = _tre(f3, pki, pkf, spb)
    return out[0] + out[32]
